# Initial kernel scaffold; baseline (speedup 1.0000x reference)
import jax
import jax.numpy as jnp
from jax.experimental import pallas as pl


def kernel(x, table, W1, b1, W2, b2, Wh, bh):
    raise NotImplementedError("write your pallas kernel here")


# speedup vs baseline: 1.9547x; 1.9547x over previous
import functools

import jax
import jax.numpy as jnp
from jax import lax
from jax.experimental import pallas as pl
from jax.experimental.pallas import tpu as pltpu
from jax.experimental.pallas import tpu_sc as plsc

_NC = 2
_NS = 16
_NW = _NC * _NS

_V = 100
_D = 100
_RP = 128
_CHUNK = 256
_NSEG = 7


def _mlp_body(tab_ref, w1_ref, b1_ref, w2_ref, b2_ref, wh_ref, bh_ref, out_ref):
    h = jnp.dot(tab_ref[...], w1_ref[...], precision=lax.Precision.HIGHEST)
    h = jnp.maximum(h + b1_ref[...], 0.0)
    h = jnp.dot(h, w2_ref[...], precision=lax.Precision.HIGHEST) + b2_ref[...]
    out_ref[...] = (
        jnp.dot(h, wh_ref[...], precision=lax.Precision.HIGHEST) + bh_ref[...]
    )


def _mlp_table(table, W1, b1, W2, b2, Wh, bh):
    wh_pad = jnp.pad(Wh, ((0, 0), (0, _RP - _D)))
    bh_pad = jnp.pad(bh, (0, _RP - _D))
    return pl.pallas_call(
        _mlp_body,
        out_shape=jax.ShapeDtypeStruct((_V, _RP), jnp.float32),
    )(table, W1, b1.reshape(1, -1), W2, b2.reshape(1, -1), wh_pad,
      bh_pad.reshape(1, -1))


def _make_sc_gather(n_tokens):
    assert n_tokens % (_NW * 2 * _CHUNK) == 0
    per_w = n_tokens // _NW
    n_pairs = per_w // (2 * _CHUNK)
    mesh = plsc.VectorSubcoreMesh(core_axis_name="c", subcore_axis_name="s")

    @functools.partial(
        pl.kernel,
        out_type=jax.ShapeDtypeStruct((n_tokens * _D,), jnp.float32),
        mesh=mesh,
        scratch_types=[
            pltpu.VMEM((per_w,), jnp.int32),
            pltpu.VMEM((_V * _RP,), jnp.float32),
            pltpu.VMEM((_CHUNK * _D + 16,), jnp.float32),
            pltpu.VMEM((_CHUNK * _D + 16,), jnp.float32),
            pltpu.SemaphoreType.DMA,
            pltpu.SemaphoreType.DMA,
        ],
    )
    def sc_gather(idx_hbm, tab_hbm, out_hbm, idx_v, tab_v, buf_a, buf_b,
                  sem_a, sem_b):
        wid = lax.axis_index("s") * _NC + lax.axis_index("c")
        base = wid * per_w
        pltpu.sync_copy(idx_hbm.at[pl.ds(base, per_w)], idx_v)
        pltpu.sync_copy(tab_hbm, tab_v)

        def fill(buf, chunk_start):
            def grp(g, carry):
                iv = idx_v[pl.ds(chunk_start + g * 16, 16)] * _RP
                for t in range(16):
                    src = iv[t]
                    dst = g * (16 * _D) + t * _D
                    for j in range(_NSEG):
                        buf[pl.ds(dst + j * 16, 16)] = (
                            tab_v[pl.ds(src + j * 16, 16)])
                return carry

            lax.fori_loop(0, _CHUNK // 16, grp, 0, unroll=False)

        def pair(g, carry):
            c0 = (base + 2 * g * _CHUNK) * _D
            fill(buf_a, 2 * g * _CHUNK)
            cp_a = pltpu.async_copy(
                buf_a.at[pl.ds(0, _CHUNK * _D)],
                out_hbm.at[pl.ds(c0, _CHUNK * _D)], sem_a)
            fill(buf_b, (2 * g + 1) * _CHUNK)
            cp_b = pltpu.async_copy(
                buf_b.at[pl.ds(0, _CHUNK * _D)],
                out_hbm.at[pl.ds(c0 + _CHUNK * _D, _CHUNK * _D)], sem_b)
            cp_a.wait()
            cp_b.wait()
            return carry

        lax.fori_loop(0, n_pairs, pair, 0, unroll=False)

    return sc_gather


def kernel(x, table, W1, b1, W2, b2, Wh, bh):
    B, L = x.shape
    n = B * L
    out_table = _mlp_table(table, W1, b1, W2, b2, Wh, bh)
    idx = x.reshape(-1).astype(jnp.int32)
    out_flat = _make_sc_gather(n)(idx, out_table.reshape(-1))
    return out_flat.reshape(B, L, _D)


# speedup vs baseline: 2.5731x; 1.3164x over previous
import functools

import jax
import jax.numpy as jnp
from jax import lax
from jax.experimental import pallas as pl
from jax.experimental.pallas import tpu as pltpu
from jax.experimental.pallas import tpu_sc as plsc

_NC = 2
_NS = 16
_NW = _NC * _NS

_V = 100
_D = 100
_RP = 128
_CHUNK = 256
_NSEG = 7


def _mlp_body(tab_ref, w1_ref, b1_ref, w2_ref, b2_ref, wh_ref, bh_ref, out_ref):
    h = jnp.dot(tab_ref[...], w1_ref[...], precision=lax.Precision.HIGHEST)
    h = jnp.maximum(h + b1_ref[...], 0.0)
    h = jnp.dot(h, w2_ref[...], precision=lax.Precision.HIGHEST) + b2_ref[...]
    out_ref[...] = (
        jnp.dot(h, wh_ref[...], precision=lax.Precision.HIGHEST) + bh_ref[...]
    )


def _mlp_table(table, W1, b1, W2, b2, Wh, bh):
    wh_pad = jnp.pad(Wh, ((0, 0), (0, _RP - _D)))
    bh_pad = jnp.pad(bh, (0, _RP - _D))
    return pl.pallas_call(
        _mlp_body,
        out_shape=jax.ShapeDtypeStruct((_V, _RP), jnp.float32),
    )(table, W1, b1.reshape(1, -1), W2, b2.reshape(1, -1), wh_pad,
      bh_pad.reshape(1, -1))


def _make_sc_gather(n_tokens):
    assert n_tokens % (_NW * 2 * _CHUNK) == 0
    per_w = n_tokens // _NW
    n_pairs = per_w // (2 * _CHUNK)
    mesh = plsc.VectorSubcoreMesh(core_axis_name="c", subcore_axis_name="s")

    @functools.partial(
        pl.kernel,
        out_type=jax.ShapeDtypeStruct((n_tokens * _D,), jnp.float32),
        mesh=mesh,
        scratch_types=[
            pltpu.VMEM((per_w,), jnp.int32),
            pltpu.VMEM((_V * _RP,), jnp.float32),
            pltpu.VMEM((_CHUNK * _D + 16,), jnp.float32),
            pltpu.VMEM((_CHUNK * _D + 16,), jnp.float32),
            pltpu.SemaphoreType.DMA,
            pltpu.SemaphoreType.DMA,
        ],
    )
    def sc_gather(idx_hbm, tab_hbm, out_hbm, idx_v, tab_v, buf_a, buf_b,
                  sem_a, sem_b):
        wid = lax.axis_index("s") * _NC + lax.axis_index("c")
        base = wid * per_w
        pltpu.sync_copy(idx_hbm.at[pl.ds(base, per_w)], idx_v)
        pltpu.sync_copy(tab_hbm, tab_v)

        def fill(buf, chunk_start):
            def grp(g, carry):
                iv = idx_v[pl.ds(chunk_start + g * 16, 16)] * _RP
                for t in range(16):
                    src = iv[t]
                    dst = g * (16 * _D) + t * _D
                    vals = [tab_v[pl.ds(src + j * 16, 16)]
                            for j in range(_NSEG)]
                    for j in range(_NSEG):
                        buf[pl.ds(dst + j * 16, 16)] = vals[j]
                return carry

            lax.fori_loop(0, _CHUNK // 16, grp, 0, unroll=False)

        def pair(g, carry):
            c0 = (base + 2 * g * _CHUNK) * _D
            fill(buf_a, 2 * g * _CHUNK)
            cp_a = pltpu.async_copy(
                buf_a.at[pl.ds(0, _CHUNK * _D)],
                out_hbm.at[pl.ds(c0, _CHUNK * _D)], sem_a)
            fill(buf_b, (2 * g + 1) * _CHUNK)
            cp_b = pltpu.async_copy(
                buf_b.at[pl.ds(0, _CHUNK * _D)],
                out_hbm.at[pl.ds(c0 + _CHUNK * _D, _CHUNK * _D)], sem_b)
            cp_a.wait()
            cp_b.wait()
            return carry

        lax.fori_loop(0, n_pairs, pair, 0, unroll=False)

    return sc_gather


def kernel(x, table, W1, b1, W2, b2, Wh, bh):
    B, L = x.shape
    n = B * L
    out_table = _mlp_table(table, W1, b1, W2, b2, Wh, bh)
    idx = x.reshape(-1).astype(jnp.int32)
    out_flat = _make_sc_gather(n)(idx, out_table.reshape(-1))
    return out_flat.reshape(B, L, _D)


# speedup vs baseline: 2.6255x; 1.0203x over previous
import functools

import jax
import jax.numpy as jnp
from jax import lax
from jax.experimental import pallas as pl
from jax.experimental.pallas import tpu as pltpu
from jax.experimental.pallas import tpu_sc as plsc

_NC = 2
_NS = 16
_NW = _NC * _NS

_V = 100
_D = 100
_RP = 128
_CHUNK = 256
_NSEG = 7


def _mlp_body(tab_ref, w1_ref, b1_ref, w2_ref, b2_ref, wh_ref, bh_ref, out_ref):
    h = jnp.dot(tab_ref[...], w1_ref[...], precision=lax.Precision.HIGHEST)
    h = jnp.maximum(h + b1_ref[...], 0.0)
    h = jnp.dot(h, w2_ref[...], precision=lax.Precision.HIGHEST) + b2_ref[...]
    out_ref[...] = (
        jnp.dot(h, wh_ref[...], precision=lax.Precision.HIGHEST) + bh_ref[...]
    )


def _mlp_table(table, W1, b1, W2, b2, Wh, bh):
    wh_pad = jnp.pad(Wh, ((0, 0), (0, _RP - _D)))
    bh_pad = jnp.pad(bh, (0, _RP - _D))
    return pl.pallas_call(
        _mlp_body,
        out_shape=jax.ShapeDtypeStruct((_V, _RP), jnp.float32),
    )(table, W1, b1.reshape(1, -1), W2, b2.reshape(1, -1), wh_pad,
      bh_pad.reshape(1, -1))


_NBUF = 4


def _make_sc_gather(n_tokens):
    assert n_tokens % (_NW * _NBUF * _CHUNK) == 0
    per_w = n_tokens // _NW
    n_quads = per_w // (_NBUF * _CHUNK)
    mesh = plsc.VectorSubcoreMesh(core_axis_name="c", subcore_axis_name="s")

    @functools.partial(
        pl.kernel,
        out_type=jax.ShapeDtypeStruct((n_tokens * _D,), jnp.float32),
        mesh=mesh,
        scratch_types=[
            pltpu.VMEM((_NBUF * _CHUNK,), jnp.int32),
            pltpu.VMEM((_V * _RP,), jnp.float32),
            [pltpu.VMEM((_CHUNK * _D + 16,), jnp.float32)
             for _ in range(_NBUF)],
            [pltpu.SemaphoreType.DMA for _ in range(_NBUF)],
        ],
    )
    def sc_gather(idx_hbm, tab_hbm, out_hbm, idx_v, tab_v, bufs, sems):
        wid = lax.axis_index("s") * _NC + lax.axis_index("c")
        base = wid * per_w
        pltpu.sync_copy(tab_hbm, tab_v)

        def fill(buf, local_start):
            def grp(g, carry):
                iv = idx_v[pl.ds(local_start + g * 16, 16)] * _RP
                for t in range(16):
                    src = iv[t]
                    dst = g * (16 * _D) + t * _D
                    vals = [tab_v[pl.ds(src + j * 16, 16)]
                            for j in range(_NSEG)]
                    for j in range(_NSEG):
                        buf[pl.ds(dst + j * 16, 16)] = vals[j]
                return carry

            lax.fori_loop(0, _CHUNK // 16, grp, 0, unroll=False)

        def drain(b):
            pltpu.make_async_copy(
                out_hbm.at[pl.ds(0, _CHUNK * _D)],
                bufs[b].at[pl.ds(0, _CHUNK * _D)], sems[b]).wait()

        def do_quad(q, first):
            pltpu.sync_copy(
                idx_hbm.at[pl.ds(base + q * (_NBUF * _CHUNK),
                                 _NBUF * _CHUNK)], idx_v)
            for b in range(_NBUF):
                if not first:
                    drain(b)
                fill(bufs[b], b * _CHUNK)
                c0 = (base + (q * _NBUF + b) * _CHUNK) * _D
                pltpu.async_copy(
                    bufs[b].at[pl.ds(0, _CHUNK * _D)],
                    out_hbm.at[pl.ds(c0, _CHUNK * _D)], sems[b])

        do_quad(0, True)
        lax.fori_loop(1, n_quads,
                      lambda q, c: (do_quad(q, False), c)[1], 0,
                      unroll=False)
        for b in range(_NBUF):
            drain(b)

    return sc_gather


def kernel(x, table, W1, b1, W2, b2, Wh, bh):
    B, L = x.shape
    n = B * L
    out_table = _mlp_table(table, W1, b1, W2, b2, Wh, bh)
    idx = x.reshape(-1).astype(jnp.int32)
    out_flat = _make_sc_gather(n)(idx, out_table.reshape(-1))
    return out_flat.reshape(B, L, _D)


# speedup vs baseline: 6.2313x; 2.3734x over previous
import functools

import jax
import jax.numpy as jnp
from jax import lax
from jax.experimental import pallas as pl
from jax.experimental.pallas import tpu as pltpu
from jax.experimental.pallas import tpu_sc as plsc

_NC = 2
_NS = 16
_NW = _NC * _NS

_V = 100
_D = 100
_RP = 128
_CHUNK = 160
_SEG_OFF = (0, 16, 32, 48, 64, 80, 84)


def _mlp_body(tab_ref, w1_ref, b1_ref, w2_ref, b2_ref, wh_ref, bh_ref, out_ref):
    h = jnp.dot(tab_ref[...], w1_ref[...], precision=lax.Precision.HIGHEST)
    h = jnp.maximum(h + b1_ref[...], 0.0)
    h = jnp.dot(h, w2_ref[...], precision=lax.Precision.HIGHEST) + b2_ref[...]
    out_ref[...] = (
        jnp.dot(h, wh_ref[...], precision=lax.Precision.HIGHEST) + bh_ref[...]
    )


def _mlp_table(table, W1, b1, W2, b2, Wh, bh):
    wh_pad = jnp.pad(Wh, ((0, 0), (0, _RP - _D)))
    bh_pad = jnp.pad(bh, (0, _RP - _D))
    return pl.pallas_call(
        _mlp_body,
        out_shape=jax.ShapeDtypeStruct((_V, _RP), jnp.float32),
    )(table, W1, b1.reshape(1, -1), W2, b2.reshape(1, -1), wh_pad,
      bh_pad.reshape(1, -1))


_NBUF = 4


def _make_sc_gather(n_tokens):
    assert n_tokens % (_NW * _NBUF * _CHUNK) == 0
    per_w = n_tokens // _NW
    n_quads = per_w // (_NBUF * _CHUNK)
    mesh = plsc.VectorSubcoreMesh(core_axis_name="c", subcore_axis_name="s")

    @functools.partial(
        pl.kernel,
        out_type=jax.ShapeDtypeStruct((n_tokens, _D), jnp.float32),
        mesh=mesh,
        scratch_types=[
            pltpu.VMEM((_NBUF * _CHUNK,), jnp.int32),
            pltpu.VMEM((_V * _RP,), jnp.float32),
            [pltpu.VMEM((_CHUNK, _D), jnp.float32) for _ in range(_NBUF)],
            [pltpu.SemaphoreType.DMA for _ in range(_NBUF)],
        ],
    )
    def sc_gather(idx_hbm, tab_hbm, out_hbm, idx_v, tab_v, bufs, sems):
        wid = lax.axis_index("s") * _NC + lax.axis_index("c")
        base = wid * per_w
        pltpu.sync_copy(tab_hbm, tab_v)

        def fill(buf, local_start):
            def grp(g, carry):
                iv = idx_v[pl.ds(local_start + g * 16, 16)] * _RP
                for t in range(16):
                    src = iv[t]
                    vals = [tab_v[pl.ds(src + o, 16)] for o in _SEG_OFF]
                    for o, val in zip(_SEG_OFF, vals):
                        buf[g * 16 + t, pl.ds(o, 16)] = val
                return carry

            lax.fori_loop(0, _CHUNK // 16, grp, 0, unroll=False)

        def drain(b):
            pltpu.make_async_copy(
                out_hbm.at[pl.ds(0, _CHUNK)], bufs[b], sems[b]).wait()

        def do_quad(q, first):
            pltpu.sync_copy(
                idx_hbm.at[pl.ds(base + q * (_NBUF * _CHUNK),
                                 _NBUF * _CHUNK)], idx_v)
            for b in range(_NBUF):
                if not first:
                    drain(b)
                fill(bufs[b], b * _CHUNK)
                r0 = base + (q * _NBUF + b) * _CHUNK
                pltpu.async_copy(
                    bufs[b], out_hbm.at[pl.ds(r0, _CHUNK)], sems[b])

        do_quad(0, True)
        lax.fori_loop(1, n_quads,
                      lambda q, c: (do_quad(q, False), c)[1], 0,
                      unroll=False)
        for b in range(_NBUF):
            drain(b)

    return sc_gather


def kernel(x, table, W1, b1, W2, b2, Wh, bh):
    B, L = x.shape
    n = B * L
    out_table = _mlp_table(table, W1, b1, W2, b2, Wh, bh)
    idx = x.reshape(-1).astype(jnp.int32)
    out_flat = _make_sc_gather(n)(idx, out_table.reshape(-1))
    return out_flat.reshape(B, L, _D)
